# baseline (speedup 1.0000x reference)
import jax
import jax.numpy as jnp
from jax.experimental import pallas as pl
from jax.experimental.pallas import tpu as pltpu

_FEAT = 3 * 32 * 32


def _body(it_ref, cur_ref, out_ref):
    del it_ref
    out_ref[...] = jnp.tanh(cur_ref[0]) * 2.0


def kernel(curriculum, curriculum_labels_one_hot, it):
    n, b = curriculum.shape[0], curriculum.shape[1]
    img_shape = curriculum.shape[2:]
    cur = curriculum.reshape(n, b, _FEAT)
    it_arr = jnp.atleast_1d(jnp.asarray(it, jnp.int32))
    grid = 8
    bs = b // grid
    out = pl.pallas_call(
        _body,
        grid_spec=pltpu.PrefetchScalarGridSpec(
            num_scalar_prefetch=1,
            grid=(grid,),
            in_specs=[
                pl.BlockSpec((1, bs, _FEAT), lambda i, it_ref: (it_ref[0], i, 0)),
            ],
            out_specs=pl.BlockSpec((bs, _FEAT), lambda i, it_ref: (i, 0)),
        ),
        out_shape=jax.ShapeDtypeStruct((b, _FEAT), jnp.float32),
    )(it_arr, cur)
    return out.reshape((b,) + img_shape), curriculum_labels_one_hot[it]
